# Initial kernel scaffold; baseline (speedup 1.0000x reference)
import jax
import jax.numpy as jnp
from jax.experimental import pallas as pl


def kernel(drug_emb, ddi_edge_index, W, b, alpha):
    raise NotImplementedError("write your pallas kernel here")


# speedup vs baseline: 11.2394x; 11.2394x over previous
import functools

import jax
import jax.numpy as jnp
from jax import lax
from jax.experimental import pallas as pl
from jax.experimental.pallas import tpu as pltpu
from jax.experimental.pallas import tpu_sc as plsc

N_NODES = 10000
HIDDEN = 128

NC = 2
NS = 16
NW = NC * NS

CHUNK = 128
CHUNKS_PER_TILE = 80
E_PER_TILE = CHUNK * CHUNKS_PER_TILE
E_PAD = E_PER_TILE * NW
N_PAD = 10240
ROWS_PER_TILE = N_PAD // NS

_mesh = plsc.VectorSubcoreMesh(
    core_axis_name="c", subcore_axis_name="s", num_cores=NC, num_subcores=NS
)


@functools.partial(
    pl.kernel,
    out_type=jax.ShapeDtypeStruct((NC, N_PAD, HIDDEN), jnp.float32),
    mesh=_mesh,
    scratch_types=dict(
        deg_sh=pltpu.VMEM_SHARED((N_PAD, HIDDEN), jnp.float32),
        col_v=pltpu.VMEM((CHUNKS_PER_TILE, CHUNK), jnp.int32),
        ones_v=pltpu.VMEM((CHUNK, HIDDEN), jnp.float32),
    ),
)
def _deg_kernel(col_hbm, ones_hbm, zeros_hbm, deg_out, *, deg_sh, col_v, ones_v):
    c = lax.axis_index("c")
    s = lax.axis_index("s")
    wid = c * NS + s
    pltpu.sync_copy(col_hbm.at[wid], col_v)
    pltpu.sync_copy(ones_hbm, ones_v)
    r0 = s * ROWS_PER_TILE
    pltpu.sync_copy(zeros_hbm.at[pl.ds(r0, ROWS_PER_TILE)],
                    deg_sh.at[pl.ds(r0, ROWS_PER_TILE)])
    plsc.subcore_barrier()
    for j in range(CHUNKS_PER_TILE):
        pltpu.sync_copy(ones_v, deg_sh.at[col_v.at[j]], add=True)
    plsc.subcore_barrier()
    pltpu.sync_copy(deg_sh.at[pl.ds(r0, ROWS_PER_TILE)],
                    deg_out.at[c, pl.ds(r0, ROWS_PER_TILE)])


@functools.partial(
    pl.kernel,
    out_type=jax.ShapeDtypeStruct((NC, N_PAD, HIDDEN), jnp.float32),
    mesh=_mesh,
    scratch_types=dict(
        acc_sh=pltpu.VMEM_SHARED((N_PAD, HIDDEN), jnp.float32),
        ridx_v=pltpu.VMEM((CHUNKS_PER_TILE, CHUNK), jnp.int32),
        cidx_v=pltpu.VMEM((CHUNKS_PER_TILE, CHUNK), jnp.int32),
        rows_v=pltpu.VMEM((CHUNK, HIDDEN), jnp.float32),
        sem=pltpu.SemaphoreType.DMA,
    ),
)
def _scatter_kernel(row_hbm, col_hbm, xs_hbm, zeros_hbm, acc_out,
                    *, acc_sh, ridx_v, cidx_v, rows_v, sem):
    c = lax.axis_index("c")
    s = lax.axis_index("s")
    wid = c * NS + s
    pltpu.sync_copy(row_hbm.at[wid], ridx_v)
    pltpu.sync_copy(col_hbm.at[wid], cidx_v)
    r0 = s * ROWS_PER_TILE
    pltpu.sync_copy(zeros_hbm.at[pl.ds(r0, ROWS_PER_TILE)],
                    acc_sh.at[pl.ds(r0, ROWS_PER_TILE)])
    plsc.subcore_barrier()
    for j in range(CHUNKS_PER_TILE):
        pltpu.async_copy(xs_hbm.at[ridx_v.at[j]], rows_v, sem).wait()
        pltpu.sync_copy(rows_v, acc_sh.at[cidx_v.at[j]], add=True)
    plsc.subcore_barrier()
    pltpu.sync_copy(acc_sh.at[pl.ds(r0, ROWS_PER_TILE)],
                    acc_out.at[c, pl.ds(r0, ROWS_PER_TILE)])


def _xs_body(emb_ref, w_ref, dega_ref, degb_ref, xs_ref, dis_ref):
    deg = dega_ref[0, :, 0:1] + degb_ref[0, :, 0:1] + 1.0
    dis = lax.rsqrt(deg)
    x = jnp.dot(emb_ref[...], w_ref[...], preferred_element_type=jnp.float32,
                precision=lax.Precision.HIGHEST)
    xs_ref[...] = x * dis
    dis_ref[...] = dis


def _combine_body(emb_ref, xs_ref, acca_ref, accb_ref, dis_ref, b_ref,
                  alpha_ref, out_ref):
    alpha = alpha_ref[0, 0]
    comb = xs_ref[...] + acca_ref[0] + accb_ref[0]
    gcn = dis_ref[...] * comb + b_ref[...]
    out_ref[...] = alpha * emb_ref[...] + (1.0 - alpha) * gcn


_BLK = 400


def kernel(drug_emb, ddi_edge_index, W, b, alpha):
    n = N_NODES
    row = ddi_edge_index[0].astype(jnp.int32)
    col = ddi_edge_index[1].astype(jnp.int32)
    e = row.shape[0]
    e_per_tile_real = e // NW
    pad_per_tile = E_PER_TILE - e_per_tile_real
    row3 = jnp.pad(row.reshape(NW, e_per_tile_real), ((0, 0), (0, pad_per_tile)),
                   constant_values=0).reshape(NW, CHUNKS_PER_TILE, CHUNK)
    col3 = jnp.pad(col.reshape(NW, e_per_tile_real), ((0, 0), (0, pad_per_tile)),
                   constant_values=n).reshape(NW, CHUNKS_PER_TILE, CHUNK)

    onesH = jnp.ones((CHUNK, HIDDEN), jnp.float32)
    zerosH = jnp.zeros((N_PAD, HIDDEN), jnp.float32)

    deg = _deg_kernel(col3, onesH, zerosH)

    grid = n // _BLK
    xs, dis = pl.pallas_call(
        _xs_body,
        grid=(grid,),
        in_specs=[
            pl.BlockSpec((_BLK, HIDDEN), lambda j: (j, 0)),
            pl.BlockSpec((HIDDEN, HIDDEN), lambda j: (0, 0)),
            pl.BlockSpec((1, _BLK, HIDDEN), lambda j: (0, j, 0)),
            pl.BlockSpec((1, _BLK, HIDDEN), lambda j: (1, j, 0)),
        ],
        out_specs=[
            pl.BlockSpec((_BLK, HIDDEN), lambda j: (j, 0)),
            pl.BlockSpec((_BLK, 1), lambda j: (j, 0)),
        ],
        out_shape=[
            jax.ShapeDtypeStruct((n, HIDDEN), jnp.float32),
            jax.ShapeDtypeStruct((n, 1), jnp.float32),
        ],
    )(drug_emb, W, deg, deg)

    acc = _scatter_kernel(row3, col3, xs, zerosH)

    enhanced = pl.pallas_call(
        _combine_body,
        grid=(grid,),
        in_specs=[
            pl.BlockSpec((_BLK, HIDDEN), lambda j: (j, 0)),
            pl.BlockSpec((_BLK, HIDDEN), lambda j: (j, 0)),
            pl.BlockSpec((1, _BLK, HIDDEN), lambda j: (0, j, 0)),
            pl.BlockSpec((1, _BLK, HIDDEN), lambda j: (1, j, 0)),
            pl.BlockSpec((_BLK, 1), lambda j: (j, 0)),
            pl.BlockSpec((1, HIDDEN), lambda j: (0, 0)),
            pl.BlockSpec((1, 1), lambda j: (0, 0)),
        ],
        out_specs=pl.BlockSpec((_BLK, HIDDEN), lambda j: (j, 0)),
        out_shape=jax.ShapeDtypeStruct((n, HIDDEN), jnp.float32),
    )(drug_emb, xs, acc, acc, dis, b.reshape(1, HIDDEN),
      alpha.reshape(1, 1))

    return enhanced


# speedup vs baseline: 12.0585x; 1.0729x over previous
import functools

import jax
import jax.numpy as jnp
from jax import lax
from jax.experimental import pallas as pl
from jax.experimental.pallas import tpu as pltpu
from jax.experimental.pallas import tpu_sc as plsc

N_NODES = 10000
HIDDEN = 128

NC = 2
NS = 16
NW = NC * NS

CHUNK = 128
CHUNKS_PER_TILE = 80
E_PER_TILE = CHUNK * CHUNKS_PER_TILE
E_PAD = E_PER_TILE * NW
N_PAD = 10240
ROWS_PER_TILE = N_PAD // NS

_mesh = plsc.VectorSubcoreMesh(
    core_axis_name="c", subcore_axis_name="s", num_cores=NC, num_subcores=NS
)


@functools.partial(
    pl.kernel,
    out_type=jax.ShapeDtypeStruct((NC, N_PAD, HIDDEN), jnp.float32),
    mesh=_mesh,
    scratch_types=dict(
        deg_sh=pltpu.VMEM_SHARED((N_PAD, HIDDEN), jnp.float32),
        col_v=pltpu.VMEM((CHUNKS_PER_TILE, CHUNK), jnp.int32),
        ones_v=pltpu.VMEM((CHUNK, HIDDEN), jnp.float32),
    ),
)
def _deg_kernel(col_hbm, ones_hbm, zeros_hbm, deg_out, *, deg_sh, col_v, ones_v):
    c = lax.axis_index("c")
    s = lax.axis_index("s")
    wid = c * NS + s
    pltpu.sync_copy(col_hbm.at[wid], col_v)
    pltpu.sync_copy(ones_hbm, ones_v)
    r0 = s * ROWS_PER_TILE
    pltpu.sync_copy(zeros_hbm.at[pl.ds(r0, ROWS_PER_TILE)],
                    deg_sh.at[pl.ds(r0, ROWS_PER_TILE)])
    plsc.subcore_barrier()
    for j in range(CHUNKS_PER_TILE):
        pltpu.sync_copy(ones_v, deg_sh.at[col_v.at[j]], add=True)
    plsc.subcore_barrier()
    pltpu.sync_copy(deg_sh.at[pl.ds(r0, ROWS_PER_TILE)],
                    deg_out.at[c, pl.ds(r0, ROWS_PER_TILE)])


@functools.partial(
    pl.kernel,
    out_type=jax.ShapeDtypeStruct((NC, N_PAD, HIDDEN), jnp.float32),
    mesh=_mesh,
    scratch_types=dict(
        acc_sh=pltpu.VMEM_SHARED((N_PAD, HIDDEN), jnp.float32),
        ridx_v=pltpu.VMEM((CHUNKS_PER_TILE // 2, CHUNK), jnp.int32),
        cidx_v=pltpu.VMEM((CHUNKS_PER_TILE // 2, CHUNK), jnp.int32),
        rows_a=pltpu.VMEM((CHUNK, HIDDEN), jnp.float32),
        rows_b=pltpu.VMEM((CHUNK, HIDDEN), jnp.float32),
        sem_a=pltpu.SemaphoreType.DMA,
        sem_b=pltpu.SemaphoreType.DMA,
    ),
)
def _scatter_kernel(row_hbm, col_hbm, xs_hbm, zeros_hbm, acc_out,
                    *, acc_sh, ridx_v, cidx_v, rows_a, rows_b, sem_a, sem_b):
    c = lax.axis_index("c")
    s = lax.axis_index("s")
    wid = c * NS + s
    r0 = s * ROWS_PER_TILE
    pltpu.sync_copy(zeros_hbm.at[pl.ds(r0, ROWS_PER_TILE)],
                    acc_sh.at[pl.ds(r0, ROWS_PER_TILE)])
    plsc.subcore_barrier()
    bufs = (rows_a, rows_b)
    sems = (sem_a, sem_b)
    half = CHUNKS_PER_TILE // 2
    for h in range(2):
        pltpu.sync_copy(row_hbm.at[wid, pl.ds(h * half, half)], ridx_v)
        pltpu.sync_copy(col_hbm.at[wid, pl.ds(h * half, half)], cidx_v)
        descs = [pltpu.async_copy(xs_hbm.at[ridx_v.at[0]], bufs[0], sems[0])]
        for j in range(half):
            descs[j].wait()
            if j + 1 < half:
                descs.append(pltpu.async_copy(
                    xs_hbm.at[ridx_v.at[j + 1]], bufs[(j + 1) % 2],
                    sems[(j + 1) % 2]))
            pltpu.sync_copy(bufs[j % 2], acc_sh.at[cidx_v.at[j]], add=True)
    plsc.subcore_barrier()
    pltpu.sync_copy(acc_sh.at[pl.ds(r0, ROWS_PER_TILE)],
                    acc_out.at[c, pl.ds(r0, ROWS_PER_TILE)])


def _xs_body(emb_ref, w_ref, dega_ref, degb_ref, xs_ref, dis_ref):
    deg = dega_ref[0, :, 0:1] + degb_ref[0, :, 0:1] + 1.0
    dis = lax.rsqrt(deg)
    x = jnp.dot(emb_ref[...], w_ref[...], preferred_element_type=jnp.float32,
                precision=lax.Precision.HIGHEST)
    xs_ref[...] = x * dis
    dis_ref[...] = dis


def _combine_body(emb_ref, xs_ref, acca_ref, accb_ref, dis_ref, b_ref,
                  alpha_ref, out_ref):
    alpha = alpha_ref[0, 0]
    comb = xs_ref[...] + acca_ref[0] + accb_ref[0]
    gcn = dis_ref[...] * comb + b_ref[...]
    out_ref[...] = alpha * emb_ref[...] + (1.0 - alpha) * gcn


_BLK = 400


def kernel(drug_emb, ddi_edge_index, W, b, alpha):
    n = N_NODES
    row = ddi_edge_index[0].astype(jnp.int32)
    col = ddi_edge_index[1].astype(jnp.int32)
    e = row.shape[0]
    e_per_tile_real = e // NW
    pad_per_tile = E_PER_TILE - e_per_tile_real
    row3 = jnp.pad(row.reshape(NW, e_per_tile_real), ((0, 0), (0, pad_per_tile)),
                   constant_values=0).reshape(NW, CHUNKS_PER_TILE, CHUNK)
    col3 = jnp.pad(col.reshape(NW, e_per_tile_real), ((0, 0), (0, pad_per_tile)),
                   constant_values=n).reshape(NW, CHUNKS_PER_TILE, CHUNK)

    onesH = jnp.ones((CHUNK, HIDDEN), jnp.float32)
    zerosH = jnp.zeros((N_PAD, HIDDEN), jnp.float32)

    deg = _deg_kernel(col3, onesH, zerosH)

    grid = n // _BLK
    xs, dis = pl.pallas_call(
        _xs_body,
        grid=(grid,),
        in_specs=[
            pl.BlockSpec((_BLK, HIDDEN), lambda j: (j, 0)),
            pl.BlockSpec((HIDDEN, HIDDEN), lambda j: (0, 0)),
            pl.BlockSpec((1, _BLK, HIDDEN), lambda j: (0, j, 0)),
            pl.BlockSpec((1, _BLK, HIDDEN), lambda j: (1, j, 0)),
        ],
        out_specs=[
            pl.BlockSpec((_BLK, HIDDEN), lambda j: (j, 0)),
            pl.BlockSpec((_BLK, 1), lambda j: (j, 0)),
        ],
        out_shape=[
            jax.ShapeDtypeStruct((n, HIDDEN), jnp.float32),
            jax.ShapeDtypeStruct((n, 1), jnp.float32),
        ],
    )(drug_emb, W, deg, deg)

    acc = _scatter_kernel(row3, col3, xs, zerosH)

    enhanced = pl.pallas_call(
        _combine_body,
        grid=(grid,),
        in_specs=[
            pl.BlockSpec((_BLK, HIDDEN), lambda j: (j, 0)),
            pl.BlockSpec((_BLK, HIDDEN), lambda j: (j, 0)),
            pl.BlockSpec((1, _BLK, HIDDEN), lambda j: (0, j, 0)),
            pl.BlockSpec((1, _BLK, HIDDEN), lambda j: (1, j, 0)),
            pl.BlockSpec((_BLK, 1), lambda j: (j, 0)),
            pl.BlockSpec((1, HIDDEN), lambda j: (0, 0)),
            pl.BlockSpec((1, 1), lambda j: (0, 0)),
        ],
        out_specs=pl.BlockSpec((_BLK, HIDDEN), lambda j: (j, 0)),
        out_shape=jax.ShapeDtypeStruct((n, HIDDEN), jnp.float32),
    )(drug_emb, xs, acc, acc, dis, b.reshape(1, HIDDEN),
      alpha.reshape(1, 1))

    return enhanced
